# baseline (speedup 1.0000x reference)
import jax
import jax.numpy as jnp
from jax import lax
from jax.experimental import pallas as pl
from jax.experimental.pallas import tpu as pltpu

_NENT = 100000
_BATCH = 1024
_KBLK = 2048
_NSTEPS = (_NENT + _KBLK - 1) // _KBLK

_DN = (((1,), (1,)), ((), ()))


def _fused_kernel(heads_ref, tails_ref, w_eh_ref, w_et_ref,
                  rels_ref, w_r_ref, w_ri_ref,
                  b_eh_ref, b_et_ref, b_r_ref, b_ri_ref,
                  out_ref, acc_h, acc_t):
    k = pl.program_id(0)

    @pl.when(k == 0)
    def _():
        acc_h[...] = jnp.zeros_like(acc_h)
        acc_t[...] = jnp.zeros_like(acc_t)

    lane = lax.broadcasted_iota(jnp.int32, (1, _KBLK), 1)
    valid = (k * _KBLK + lane) < _NENT
    h = jnp.where(valid, heads_ref[...], 0.0)
    t = jnp.where(valid, tails_ref[...], 0.0)
    w = jnp.concatenate([w_eh_ref[...], w_et_ref[...]], axis=0)
    w = jnp.where(valid, w, 0.0)

    acc_h[...] += lax.dot_general(h, w, _DN, preferred_element_type=jnp.float32)
    acc_t[...] += lax.dot_general(t, w, _DN, preferred_element_type=jnp.float32)

    @pl.when(k == _NSTEPS - 1)
    def _():
        r = lax.dot_general(rels_ref[...], w_r_ref[...], _DN,
                            preferred_element_type=jnp.float32) + b_r_ref[...]
        ri = lax.dot_general(rels_ref[...], w_ri_ref[...], _DN,
                             preferred_element_type=jnp.float32) + b_ri_ref[...]
        hh = acc_h[:, :64] + b_eh_ref[...]
        th = acc_h[:, 64:] + b_et_ref[...]
        ht = acc_t[:, :64] + b_eh_ref[...]
        tt = acc_t[:, 64:] + b_et_ref[...]
        s1 = jnp.sum(hh * r * tt, axis=1)
        s2 = jnp.sum(ht * ri * th, axis=1)
        out_ref[...] = jnp.clip((s1 + s2) * 0.5, -20.0, 20.0)[:, None]


def kernel(heads, rels, tails, W_eh, b_eh, W_et, b_et, W_r, b_r, W_ri, b_ri):
    out = pl.pallas_call(
        _fused_kernel,
        grid=(_NSTEPS,),
        in_specs=[
            pl.BlockSpec((_BATCH, _KBLK), lambda k: (0, k)),
            pl.BlockSpec((_BATCH, _KBLK), lambda k: (0, k)),
            pl.BlockSpec((64, _KBLK), lambda k: (0, k)),
            pl.BlockSpec((64, _KBLK), lambda k: (0, k)),
            pl.BlockSpec((_BATCH, 1000), lambda k: (0, 0)),
            pl.BlockSpec((64, 1000), lambda k: (0, 0)),
            pl.BlockSpec((64, 1000), lambda k: (0, 0)),
            pl.BlockSpec((1, 64), lambda k: (0, 0)),
            pl.BlockSpec((1, 64), lambda k: (0, 0)),
            pl.BlockSpec((1, 64), lambda k: (0, 0)),
            pl.BlockSpec((1, 64), lambda k: (0, 0)),
        ],
        out_specs=pl.BlockSpec((_BATCH, 1), lambda k: (0, 0)),
        out_shape=jax.ShapeDtypeStruct((_BATCH, 1), jnp.float32),
        scratch_shapes=[pltpu.VMEM((_BATCH, 128), jnp.float32)] * 2,
    )(heads, tails, W_eh, W_et, rels, W_r, W_ri,
      b_eh[None, :], b_et[None, :], b_r[None, :], b_ri[None, :])
    return out[:, 0]
